# Initial kernel scaffold; baseline (speedup 1.0000x reference)
import jax
import jax.numpy as jnp
from jax.experimental import pallas as pl


def kernel(table, rel_index):
    raise NotImplementedError("write your pallas kernel here")


# speedup vs baseline: 5.6565x; 5.6565x over previous
import jax
import jax.numpy as jnp
from jax import lax
from jax.experimental import pallas as pl
from jax.experimental.pallas import tpu as pltpu
from jax.experimental.pallas import tpu_sc as plsc

NUM_HEADS = 32
N = 1024
TOTAL = N * N
NUM_WORKERS = 32
KPW = TOTAL // NUM_WORKERS
LANES = 16
COL_PAD = 3976


def _sc_gather_kernel(table_t_hbm, idx_hbm, out_hbm, idx_v, col_v, out_v):
    wid = lax.axis_index("s") * 2 + lax.axis_index("c")
    base = wid * KPW

    pltpu.sync_copy(idx_hbm.at[pl.ds(base, KPW)], idx_v)

    def per_head(h, carry):
        pltpu.sync_copy(table_t_hbm.at[h], col_v)

        def gather_chunk(j, c):
            off = pl.multiple_of(j * LANES, LANES)
            idxv = idx_v[pl.ds(off, LANES)]
            out_v[pl.ds(off, LANES)] = plsc.load_gather(col_v, [idxv])
            return c

        lax.fori_loop(0, KPW // LANES, gather_chunk, 0, unroll=8)

        pltpu.sync_copy(out_v, out_hbm.at[h, pl.ds(base, KPW)])
        return carry

    lax.fori_loop(0, NUM_HEADS, per_head, 0)


@jax.jit
def kernel(table, rel_index):
    table_t = jnp.zeros((NUM_HEADS, COL_PAD), jnp.float32)
    table_t = lax.dynamic_update_slice(table_t, table.T, (0, 0))
    idx_flat = rel_index.reshape(TOTAL).astype(jnp.int32)

    mesh = plsc.VectorSubcoreMesh(
        core_axis_name="c", subcore_axis_name="s", num_cores=2, num_subcores=16
    )
    out_flat = pl.kernel(
        _sc_gather_kernel,
        out_type=jax.ShapeDtypeStruct((NUM_HEADS, TOTAL), jnp.float32),
        mesh=mesh,
        compiler_params=pltpu.CompilerParams(needs_layout_passes=False),
        scratch_types=[
            pltpu.VMEM((KPW,), jnp.int32),
            pltpu.VMEM((COL_PAD,), jnp.float32),
            pltpu.VMEM((KPW,), jnp.float32),
        ],
    )(table_t, idx_flat)
    return out_flat.reshape(NUM_HEADS, N, N)


# speedup vs baseline: 6.2859x; 1.1113x over previous
import jax
import jax.numpy as jnp
from jax import lax
from jax.experimental import pallas as pl
from jax.experimental.pallas import tpu as pltpu
from jax.experimental.pallas import tpu_sc as plsc

NUM_HEADS = 32
N = 1024
TOTAL = N * N
NUM_WORKERS = 32
KPW = TOTAL // NUM_WORKERS
LANES = 16
COL_PAD = 3976


def _sc_gather_kernel(
    table_t_hbm, idx_hbm, out_hbm,
    idx_v, col0, col1, out0, out1, col_sem, out_sem,
):
    wid = lax.axis_index("s") * 2 + lax.axis_index("c")
    base = wid * KPW

    pltpu.sync_copy(idx_hbm.at[pl.ds(base, KPW)], idx_v)

    cols = [col0, col1]
    outs = [out0, out1]
    col_copies = {0: pltpu.async_copy(table_t_hbm.at[0], col0, col_sem)}
    out_copies = {}

    for h in range(NUM_HEADS):
        col_copies.pop(h).wait()
        if h + 1 < NUM_HEADS:
            col_copies[h + 1] = pltpu.async_copy(
                table_t_hbm.at[h + 1], cols[(h + 1) % 2], col_sem
            )
        if h >= 2:
            out_copies.pop(h - 2).wait()
        col_v = cols[h % 2]
        out_v = outs[h % 2]

        def gather_chunk(j, c, col_v=col_v, out_v=out_v):
            off = pl.multiple_of(j * LANES, LANES)
            idxv = idx_v[pl.ds(off, LANES)]
            out_v[pl.ds(off, LANES)] = plsc.load_gather(col_v, [idxv])
            return c

        lax.fori_loop(0, KPW // LANES, gather_chunk, 0, unroll=8)

        out_copies[h] = pltpu.async_copy(
            out_v, out_hbm.at[h, pl.ds(base, KPW)], out_sem
        )

    out_copies.pop(NUM_HEADS - 2).wait()
    out_copies.pop(NUM_HEADS - 1).wait()


@jax.jit
def kernel(table, rel_index):
    table_t = jnp.zeros((NUM_HEADS, COL_PAD), jnp.float32)
    table_t = lax.dynamic_update_slice(table_t, table.T, (0, 0))
    idx_flat = rel_index.reshape(TOTAL).astype(jnp.int32)

    mesh = plsc.VectorSubcoreMesh(
        core_axis_name="c", subcore_axis_name="s", num_cores=2, num_subcores=16
    )
    out_flat = pl.kernel(
        _sc_gather_kernel,
        out_type=jax.ShapeDtypeStruct((NUM_HEADS, TOTAL), jnp.float32),
        mesh=mesh,
        compiler_params=pltpu.CompilerParams(needs_layout_passes=False),
        scratch_types=[
            pltpu.VMEM((KPW,), jnp.int32),
            pltpu.VMEM((COL_PAD,), jnp.float32),
            pltpu.VMEM((COL_PAD,), jnp.float32),
            pltpu.VMEM((KPW,), jnp.float32),
            pltpu.VMEM((KPW,), jnp.float32),
            pltpu.SemaphoreType.DMA,
            pltpu.SemaphoreType.DMA,
        ],
    )(table_t, idx_flat)
    return out_flat.reshape(NUM_HEADS, N, N)


# speedup vs baseline: 19.1429x; 3.0454x over previous
import jax
import jax.numpy as jnp
from jax import lax
from jax.experimental import pallas as pl
from jax.experimental.pallas import tpu as pltpu
from jax.experimental.pallas import tpu_sc as plsc

NUM_HEADS = 32
N = 1024
TOTAL = N * N
NUM_WORKERS = 32
KPW = TOTAL // NUM_WORKERS
LANES = 16
COL_PAD = 3976


def _sc_gather_kernel(
    table_t_hbm, idx_hbm, out_hbm,
    idx_v, col0, col1, out0, out1, col_sem, out_sem,
):
    wid = lax.axis_index("s") * 2 + lax.axis_index("c")
    base = wid * KPW

    pltpu.sync_copy(idx_hbm.at[pl.ds(base, KPW)], idx_v)

    cols = [col0, col1]
    outs = [out0, out1]
    col_copies = {0: pltpu.async_copy(table_t_hbm.at[0], col0, col_sem)}
    out_copies = {}

    for h in range(NUM_HEADS):
        col_copies.pop(h).wait()
        if h + 1 < NUM_HEADS:
            col_copies[h + 1] = pltpu.async_copy(
                table_t_hbm.at[h + 1], cols[(h + 1) % 2], col_sem
            )
        if h >= 2:
            out_copies.pop(h - 2).wait()
        col_v = cols[h % 2]
        out_v = outs[h % 2]

        @plsc.parallel_loop(0, KPW, step=LANES, unroll=8)
        def gather_chunk(off, col_v=col_v, out_v=out_v):
            idxv = idx_v[pl.ds(off, LANES)]
            out_v[pl.ds(off, LANES)] = plsc.load_gather(col_v, [idxv])

        out_copies[h] = pltpu.async_copy(
            out_v, out_hbm.at[h, pl.ds(base, KPW)], out_sem
        )

    out_copies.pop(NUM_HEADS - 2).wait()
    out_copies.pop(NUM_HEADS - 1).wait()


@jax.jit
def kernel(table, rel_index):
    table_t = jnp.zeros((NUM_HEADS, COL_PAD), jnp.float32)
    table_t = lax.dynamic_update_slice(table_t, table.T, (0, 0))
    idx_flat = rel_index.reshape(TOTAL).astype(jnp.int32)

    mesh = plsc.VectorSubcoreMesh(
        core_axis_name="c", subcore_axis_name="s", num_cores=2, num_subcores=16
    )
    out_flat = pl.kernel(
        _sc_gather_kernel,
        out_type=jax.ShapeDtypeStruct((NUM_HEADS, TOTAL), jnp.float32),
        mesh=mesh,
        compiler_params=pltpu.CompilerParams(needs_layout_passes=False),
        scratch_types=[
            pltpu.VMEM((KPW,), jnp.int32),
            pltpu.VMEM((COL_PAD,), jnp.float32),
            pltpu.VMEM((COL_PAD,), jnp.float32),
            pltpu.VMEM((KPW,), jnp.float32),
            pltpu.VMEM((KPW,), jnp.float32),
            pltpu.SemaphoreType.DMA,
            pltpu.SemaphoreType.DMA,
        ],
    )(table_t, idx_flat)
    return out_flat.reshape(NUM_HEADS, N, N)


# speedup vs baseline: 34.4468x; 1.7995x over previous
import jax
import jax.numpy as jnp
from jax import lax
from jax.experimental import pallas as pl
from jax.experimental.pallas import tpu as pltpu
from jax.experimental.pallas import tpu_sc as plsc

NUM_HEADS = 32
N = 1024
NUM_WORKERS = 32
ROWS_PW = N // NUM_WORKERS
KPW = ROWS_PW * N
LANES = 16
COL_PAD = 3976


def _sc_gather_kernel(
    table_t_hbm, idx_hbm, out_hbm,
    idx_v, col0, col1, out0, out1, col_sem, out_sem,
):
    wid = lax.axis_index("s") * 2 + lax.axis_index("c")
    row_base = wid * ROWS_PW

    pltpu.sync_copy(idx_hbm.at[pl.ds(row_base, ROWS_PW)], idx_v)

    cols = [col0, col1]
    outs = [out0, out1]
    col_copies = {0: pltpu.async_copy(table_t_hbm.at[0], col0, col_sem)}
    out_copies = {}

    for h in range(NUM_HEADS):
        col_copies.pop(h).wait()
        if h + 1 < NUM_HEADS:
            col_copies[h + 1] = pltpu.async_copy(
                table_t_hbm.at[h + 1], cols[(h + 1) % 2], col_sem
            )
        if h >= 2:
            out_copies.pop(h - 2).wait()
        col_v = cols[h % 2]
        out_v = outs[h % 2]

        @plsc.parallel_loop(0, KPW, step=LANES, unroll=8)
        def gather_chunk(off, col_v=col_v, out_v=out_v):
            r = off // N
            c = off % N
            idxv = idx_v[r, pl.ds(c, LANES)]
            out_v[r, pl.ds(c, LANES)] = plsc.load_gather(col_v, [idxv])

        out_copies[h] = pltpu.async_copy(
            out_v, out_hbm.at[h, pl.ds(row_base, ROWS_PW)], out_sem
        )

    out_copies.pop(NUM_HEADS - 2).wait()
    out_copies.pop(NUM_HEADS - 1).wait()


@jax.jit
def kernel(table, rel_index):
    table_t = jnp.zeros((NUM_HEADS, COL_PAD), jnp.float32)
    table_t = lax.dynamic_update_slice(table_t, table.T, (0, 0))
    idx = rel_index.astype(jnp.int32)

    mesh = plsc.VectorSubcoreMesh(
        core_axis_name="c", subcore_axis_name="s", num_cores=2, num_subcores=16
    )
    return pl.kernel(
        _sc_gather_kernel,
        out_type=jax.ShapeDtypeStruct((NUM_HEADS, N, N), jnp.float32),
        mesh=mesh,
        compiler_params=pltpu.CompilerParams(needs_layout_passes=False),
        scratch_types=[
            pltpu.VMEM((ROWS_PW, N), jnp.int32),
            pltpu.VMEM((COL_PAD,), jnp.float32),
            pltpu.VMEM((COL_PAD,), jnp.float32),
            pltpu.VMEM((ROWS_PW, N), jnp.float32),
            pltpu.VMEM((ROWS_PW, N), jnp.float32),
            pltpu.SemaphoreType.DMA,
            pltpu.SemaphoreType.DMA,
        ],
    )(table_t, idx)


# speedup vs baseline: 39.3455x; 1.1422x over previous
import jax
import jax.numpy as jnp
from jax import lax
from jax.experimental import pallas as pl
from jax.experimental.pallas import tpu as pltpu
from jax.experimental.pallas import tpu_sc as plsc

NUM_HEADS = 32
N = 1024
NUM_WORKERS = 32
ROWS_PW = N // NUM_WORKERS
LANES = 16
COL_PAD = 3976
G = 4
NG = NUM_HEADS // G
SB_ROWS = 4
NSB = ROWS_PW // SB_ROWS
SB_ELEMS = SB_ROWS * N


def _sc_gather_kernel(
    table_t_hbm, idx_hbm, out_hbm,
    idx_v, cg00, cg01, cg02, cg03, cg10, cg11, cg12, cg13,
    ob00, ob01, ob02, ob03, ob10, ob11, ob12, ob13,
    col_sem, out_sem,
):
    wid = lax.axis_index("s") * 2 + lax.axis_index("c")
    row_base = wid * ROWS_PW

    pltpu.sync_copy(idx_hbm.at[pl.ds(row_base, ROWS_PW)], idx_v)

    colgs = [[cg00, cg01, cg02, cg03], [cg10, cg11, cg12, cg13]]
    outbs = [[ob00, ob01, ob02, ob03], [ob10, ob11, ob12, ob13]]

    def start_col_group(g):
        return [
            pltpu.async_copy(table_t_hbm.at[g * G + hd], colgs[g % 2][hd], col_sem)
            for hd in range(G)
        ]

    col_copies = {0: start_col_group(0)}
    pending_out = {0: [], 1: []}

    for g in range(NG):
        for cp in col_copies.pop(g):
            cp.wait()
        if g + 1 < NG:
            col_copies[g + 1] = start_col_group(g + 1)
        col_v = colgs[g % 2]

        for sb in range(NSB):
            par = sb % 2
            buf = outbs[par]
            for cp in pending_out[par]:
                cp.wait()
            pending_out[par] = []

            @plsc.parallel_loop(0, SB_ELEMS, step=LANES, unroll=4)
            def gather_chunk(off, col_v=col_v, buf=buf, sb=sb):
                r = off // N
                c = off % N
                idxv = idx_v[sb * SB_ROWS + r, pl.ds(c, LANES)]
                for hd in range(G):
                    buf[hd][r, pl.ds(c, LANES)] = plsc.load_gather(
                        col_v[hd], [idxv]
                    )

            for hd in range(G):
                pending_out[par].append(
                    pltpu.async_copy(
                        buf[hd],
                        out_hbm.at[g * G + hd, pl.ds(row_base + sb * SB_ROWS, SB_ROWS)],
                        out_sem,
                    )
                )

    for par in (0, 1):
        for cp in pending_out[par]:
            cp.wait()


@jax.jit
def kernel(table, rel_index):
    table_t = jnp.zeros((NUM_HEADS, COL_PAD), jnp.float32)
    table_t = lax.dynamic_update_slice(table_t, table.T, (0, 0))
    idx = rel_index.astype(jnp.int32)

    mesh = plsc.VectorSubcoreMesh(
        core_axis_name="c", subcore_axis_name="s", num_cores=2, num_subcores=16
    )
    return pl.kernel(
        _sc_gather_kernel,
        out_type=jax.ShapeDtypeStruct((NUM_HEADS, N, N), jnp.float32),
        mesh=mesh,
        compiler_params=pltpu.CompilerParams(needs_layout_passes=False),
        scratch_types=[
            pltpu.VMEM((ROWS_PW, N), jnp.int32),
            pltpu.VMEM((COL_PAD,), jnp.float32),
            pltpu.VMEM((COL_PAD,), jnp.float32),
            pltpu.VMEM((COL_PAD,), jnp.float32),
            pltpu.VMEM((COL_PAD,), jnp.float32),
            pltpu.VMEM((COL_PAD,), jnp.float32),
            pltpu.VMEM((COL_PAD,), jnp.float32),
            pltpu.VMEM((COL_PAD,), jnp.float32),
            pltpu.VMEM((COL_PAD,), jnp.float32),
            pltpu.VMEM((SB_ROWS, N), jnp.float32),
            pltpu.VMEM((SB_ROWS, N), jnp.float32),
            pltpu.VMEM((SB_ROWS, N), jnp.float32),
            pltpu.VMEM((SB_ROWS, N), jnp.float32),
            pltpu.VMEM((SB_ROWS, N), jnp.float32),
            pltpu.VMEM((SB_ROWS, N), jnp.float32),
            pltpu.VMEM((SB_ROWS, N), jnp.float32),
            pltpu.VMEM((SB_ROWS, N), jnp.float32),
            pltpu.SemaphoreType.DMA,
            pltpu.SemaphoreType.DMA,
        ],
    )(table_t, idx)
